# baseline (speedup 1.0000x reference)
import functools

import jax
import jax.numpy as jnp
from jax import lax
from jax.experimental import pallas as pl
from jax.experimental.pallas import tpu as pltpu
from jax.experimental.pallas import tpu_sc as plsc

CLS = 1_000_000
BATCH = 16384
FEAT = 32
NC = 2
NS = 16
NW = NC * NS
BPW = BATCH // NW
ROWS = BATCH // 128
CROWS = ROWS // NS
PROWS = ROWS // NW


def _sc_body(label2d, centers, scent_out, counts_out,
             idx_c, idx_p, val_v, counts_v, scent_v, hist, sem):
    c = lax.axis_index("c")
    s = lax.axis_index("s")
    wid = s * NC + c

    pltpu.sync_copy(label2d.at[pl.ds(s * CROWS, CROWS)], idx_c)

    for i in range(128 // 16):
        val_v[pl.ds(i * 16, 16)] = jnp.zeros((16,), jnp.float32)
    for j in range(CROWS):
        pltpu.sync_copy(val_v, hist.at[idx_c.at[j]])
    plsc.subcore_barrier()

    for i in range(128 // 16):
        val_v[pl.ds(i * 16, 16)] = jnp.ones((16,), jnp.float32)
    for j in range(CROWS):
        pltpu.sync_copy(val_v, hist.at[idx_c.at[j]], add=True)
    plsc.subcore_barrier()

    pltpu.sync_copy(label2d.at[pl.ds(wid * PROWS, PROWS)], idx_p)
    for j in range(PROWS):
        pltpu.sync_copy(hist.at[idx_p.at[j]], counts_v.at[j])
        pltpu.async_copy(centers.at[idx_p.at[j]],
                         scent_v.at[pl.ds(j * 128, 128)], sem).wait()

    pltpu.sync_copy(scent_v, scent_out.at[pl.ds(wid * BPW, BPW)])
    pltpu.sync_copy(counts_v, counts_out.at[pl.ds(wid * PROWS, PROWS)])


@jax.jit
def _sc_gather(label2d, centers):
    mesh = plsc.VectorSubcoreMesh(core_axis_name="c", subcore_axis_name="s")
    return pl.kernel(
        _sc_body,
        out_type=[
            jax.ShapeDtypeStruct((BATCH, FEAT), jnp.float32),
            jax.ShapeDtypeStruct((ROWS, 128), jnp.float32),
        ],
        mesh=mesh,
        scratch_types=[
            pltpu.VMEM((CROWS, 128), jnp.int32),
            pltpu.VMEM((PROWS, 128), jnp.int32),
            pltpu.VMEM((128,), jnp.float32),
            pltpu.VMEM((PROWS, 128), jnp.float32),
            pltpu.VMEM((BPW, FEAT), jnp.float32),
            pltpu.VMEM_SHARED((CLS,), jnp.float32),
            pltpu.SemaphoreType.DMA,
        ],
        compiler_params=pltpu.CompilerParams(use_tc_tiling_on_sc=False),
    )(label2d, centers)


def _tc_loss_body(feat_ref, scent_ref, counts_ref, out_ref):
    d = feat_ref[...] - scent_ref[...]
    sq = jnp.sum(d * d, axis=1, keepdims=True)
    t = sq / counts_ref[...]
    out_ref[0, 0] = jnp.sum(jnp.sqrt(t)) / BATCH


@jax.jit
def _tc_loss(feat, scent, counts):
    return pl.pallas_call(
        _tc_loss_body,
        out_shape=jax.ShapeDtypeStruct((1, 1), jnp.float32),
        in_specs=[pl.BlockSpec(memory_space=pltpu.VMEM)] * 3,
        out_specs=pl.BlockSpec(memory_space=pltpu.SMEM),
    )(feat, scent, counts)


def kernel(feat, label, centers):
    label2d = label.astype(jnp.int32).reshape(ROWS, 128)
    scent, counts = _sc_gather(label2d, centers)
    loss = _tc_loss(feat, scent, counts.reshape(BATCH, 1))
    return loss[0, 0]


# speedup vs baseline: 1.0124x; 1.0124x over previous
import functools

import jax
import jax.numpy as jnp
from jax import lax
from jax.experimental import pallas as pl
from jax.experimental.pallas import tpu as pltpu
from jax.experimental.pallas import tpu_sc as plsc

CLS = 1_000_000
BATCH = 16384
FEAT = 32
NC = 2
NS = 16
NW = NC * NS
BPW = BATCH // NW
ROWS = BATCH // 128
CROWS = ROWS // NS
PROWS = ROWS // NW
GROUPS = BPW // 16


def _sc_body(label2d, feat4, cent128, t_out,
             idx_c, idx_p, widx, val_v, counts_v, feat_v, cent_v, t_v,
             hist, sem_g, sem_g2, sem_f, sem_s):
    c = lax.axis_index("c")
    s = lax.axis_index("s")
    wid = s * NC + c

    pltpu.sync_copy(label2d.at[pl.ds(s * CROWS, CROWS)], idx_c)
    pltpu.sync_copy(label2d.at[pl.ds(wid * PROWS, PROWS)], idx_p)

    for j in range(PROWS):
        for i in range(128 // 16):
            widx[j, pl.ds(i * 16, 16)] = (
                idx_p[j, pl.ds(i * 16, 16)] >> 2)

    gsems = [sem_g, sem_g2]
    gcps = [pltpu.async_copy(cent128.at[widx.at[j]],
                             cent_v.at[pl.ds((j % 2) * 128, 128)], gsems[j % 2])
            for j in range(2)]
    fcp = pltpu.async_copy(feat4.at[pl.ds(wid * 128, 128)], feat_v, sem_f)

    for i in range(128 // 16):
        val_v[pl.ds(i * 16, 16)] = jnp.zeros((16,), jnp.float32)
    zcps = [pltpu.async_copy(val_v, hist.at[idx_c.at[j]], sem_s)
            for j in range(CROWS)]
    for cp in zcps:
        cp.wait()
    plsc.subcore_barrier()

    for i in range(128 // 16):
        val_v[pl.ds(i * 16, 16)] = jnp.ones((16,), jnp.float32)
    acps = [pltpu.async_copy(val_v, hist.at[idx_c.at[j]], sem_s, add=True)
            for j in range(CROWS)]
    for cp in acps:
        cp.wait()
    plsc.subcore_barrier()

    ccps = [pltpu.async_copy(hist.at[idx_p.at[j]], counts_v.at[j], sem_s)
            for j in range(PROWS)]
    for cp in ccps:
        cp.wait()
    fcp.wait()

    lane = lax.iota(jnp.int32, 16)

    for cj in range(PROWS):
        gcps[cj % 2].wait()

        def group_body(gg, _, cj=cj):
            jrow = cj
            lbase = gg * 16
            buf = (cj % 2) * 128
            labels16 = idx_p[jrow, pl.ds(lbase, 16)]
            sub = (labels16 & 3) * FEAT
            prow = gg * 16 + lane
            acc = jnp.zeros((16,), jnp.float32)
            fbase = (cj * 128 + prow) * FEAT
            for f in range(FEAT):
                fv = plsc.load_gather(
                    feat_v, [(fbase + f) >> 7, (fbase + f) & 127])
                cv = plsc.load_gather(cent_v, [buf + prow, sub + f])
                d = fv - cv
                acc = acc + d * d
            cnt = counts_v[jrow, pl.ds(lbase, 16)]
            t_v[jrow, pl.ds(lbase, 16)] = acc / cnt
            return _

        lax.fori_loop(0, 8, group_body, 0, unroll=False)
        if cj + 2 < PROWS:
            gcps[(cj + 2) % 2] = pltpu.async_copy(
                cent128.at[widx.at[cj + 2]],
                cent_v.at[pl.ds(((cj + 2) % 2) * 128, 128)], gsems[(cj + 2) % 2])

    pltpu.sync_copy(t_v, t_out.at[pl.ds(wid * PROWS, PROWS)])


@jax.jit
def _sc_part(label2d, feat4, cent128):
    mesh = plsc.VectorSubcoreMesh(core_axis_name="c", subcore_axis_name="s")
    return pl.kernel(
        _sc_body,
        out_type=jax.ShapeDtypeStruct((ROWS, 128), jnp.float32),
        mesh=mesh,
        scratch_types=[
            pltpu.VMEM((CROWS, 128), jnp.int32),
            pltpu.VMEM((PROWS, 128), jnp.int32),
            pltpu.VMEM((PROWS, 128), jnp.int32),
            pltpu.VMEM((128,), jnp.float32),
            pltpu.VMEM((PROWS, 128), jnp.float32),
            pltpu.VMEM((128, 128), jnp.float32),
            pltpu.VMEM((256, 128), jnp.float32),
            pltpu.VMEM((PROWS, 128), jnp.float32),
            pltpu.VMEM_SHARED((CLS,), jnp.float32),
            pltpu.SemaphoreType.DMA,
            pltpu.SemaphoreType.DMA,
            pltpu.SemaphoreType.DMA,
            pltpu.SemaphoreType.DMA,
        ],
        compiler_params=pltpu.CompilerParams(needs_layout_passes=False),
    )(label2d, feat4, cent128)


def _tc_loss_body(t_ref, out_ref):
    out_ref[0, 0] = jnp.sum(jnp.sqrt(t_ref[...])) / BATCH


@jax.jit
def _tc_loss(t):
    return pl.pallas_call(
        _tc_loss_body,
        out_shape=jax.ShapeDtypeStruct((1, 1), jnp.float32),
        in_specs=[pl.BlockSpec(memory_space=pltpu.VMEM)],
        out_specs=pl.BlockSpec(memory_space=pltpu.SMEM),
    )(t)


def kernel(feat, label, centers):
    label2d = label.astype(jnp.int32).reshape(ROWS, 128)
    feat4 = feat.reshape(BATCH * FEAT // 128, 128)
    cent128 = centers.reshape(CLS * FEAT // 128, 128)
    t = _sc_part(label2d, feat4, cent128)
    return _tc_loss(t)[0, 0]


# speedup vs baseline: 10.3800x; 10.2533x over previous
import functools

import jax
import jax.numpy as jnp
from jax import lax
from jax.experimental import pallas as pl
from jax.experimental.pallas import tpu as pltpu
from jax.experimental.pallas import tpu_sc as plsc

CLS = 1_000_000
BATCH = 16384
FEAT = 32
NC = 2
NS = 16
NW = NC * NS
BPW = BATCH // NW
ROWS = BATCH // 128
CROWS = ROWS // NS
PROWS = ROWS // NW
GROUPS = BPW // 16


def _sc_body(label2d, featT, scentT, t_out,
             idx_c, idx_p, val_v, counts_v, featT_v, scentT_v, t_v,
             hist, sem_f, sem_g, sem_s):
    c = lax.axis_index("c")
    s = lax.axis_index("s")
    wid = s * NC + c
    base = wid * BPW

    pltpu.sync_copy(label2d.at[pl.ds(s * CROWS, CROWS)], idx_c)
    pltpu.sync_copy(label2d.at[pl.ds(wid * PROWS, PROWS)], idx_p)

    fcp = pltpu.async_copy(featT.at[:, pl.ds(base, BPW)], featT_v, sem_f)
    gcp = pltpu.async_copy(scentT.at[:, pl.ds(base, BPW)], scentT_v, sem_g)

    for i in range(128 // 16):
        val_v[pl.ds(i * 16, 16)] = jnp.zeros((16,), jnp.float32)
    zcps = [pltpu.async_copy(val_v, hist.at[idx_c.at[j]], sem_s)
            for j in range(CROWS)]
    for cp in zcps:
        cp.wait()
    plsc.subcore_barrier()

    for i in range(128 // 16):
        val_v[pl.ds(i * 16, 16)] = jnp.ones((16,), jnp.float32)
    acps = [pltpu.async_copy(val_v, hist.at[idx_c.at[j]], sem_s, add=True)
            for j in range(CROWS)]
    for cp in acps:
        cp.wait()
    plsc.subcore_barrier()

    ccps = [pltpu.async_copy(hist.at[idx_p.at[j]], counts_v.at[j], sem_s)
            for j in range(PROWS)]
    for cp in ccps:
        cp.wait()
    fcp.wait()
    gcp.wait()

    def group_body(g, _):
        jrow = g // 8
        lbase = (g % 8) * 16
        acc = jnp.zeros((16,), jnp.float32)
        for f in range(FEAT):
            a = featT_v[f, pl.ds(g * 16, 16)]
            b = scentT_v[f, pl.ds(g * 16, 16)]
            d = a - b
            acc = acc + d * d
        cnt = counts_v[jrow, pl.ds(lbase, 16)]
        t_v[jrow, pl.ds(lbase, 16)] = acc / cnt
        return _

    lax.fori_loop(0, GROUPS, group_body, 0, unroll=False)

    pltpu.sync_copy(t_v, t_out.at[pl.ds(wid * PROWS, PROWS)])


@jax.jit
def _sc_part(label2d, featT, scentT):
    mesh = plsc.VectorSubcoreMesh(core_axis_name="c", subcore_axis_name="s")
    return pl.kernel(
        _sc_body,
        out_type=jax.ShapeDtypeStruct((ROWS, 128), jnp.float32),
        mesh=mesh,
        scratch_types=[
            pltpu.VMEM((CROWS, 128), jnp.int32),
            pltpu.VMEM((PROWS, 128), jnp.int32),
            pltpu.VMEM((128,), jnp.float32),
            pltpu.VMEM((PROWS, 128), jnp.float32),
            pltpu.VMEM((FEAT, BPW), jnp.float32),
            pltpu.VMEM((FEAT, BPW), jnp.float32),
            pltpu.VMEM((PROWS, 128), jnp.float32),
            pltpu.VMEM_SHARED((CLS,), jnp.float32),
            pltpu.SemaphoreType.DMA,
            pltpu.SemaphoreType.DMA,
            pltpu.SemaphoreType.DMA,
        ],
        compiler_params=pltpu.CompilerParams(needs_layout_passes=False),
    )(label2d, featT, scentT)


def _tc_loss_body(t_ref, out_ref):
    out_ref[0, 0] = jnp.sum(jnp.sqrt(t_ref[...])) / BATCH


@jax.jit
def _tc_loss(t):
    return pl.pallas_call(
        _tc_loss_body,
        out_shape=jax.ShapeDtypeStruct((1, 1), jnp.float32),
        in_specs=[pl.BlockSpec(memory_space=pltpu.VMEM)],
        out_specs=pl.BlockSpec(memory_space=pltpu.SMEM),
    )(t)


def kernel(feat, label, centers):
    label = label.astype(jnp.int32)
    label2d = label.reshape(ROWS, 128)
    scent = jnp.take(centers, label, axis=0)
    t = _sc_part(label2d, feat.T, scent.T)
    return _tc_loss(t)[0, 0]
